# Initial kernel scaffold; baseline (speedup 1.0000x reference)
import jax
import jax.numpy as jnp
from jax.experimental import pallas as pl


def kernel(relative_position_bias_table, cls_to_patches, patches_to_cls, cls_to_cls, relative_position_index):
    raise NotImplementedError("write your pallas kernel here")


# speedup vs baseline: 4.9890x; 4.9890x over previous
import jax
import jax.numpy as jnp
from jax import lax
from jax.experimental import pallas as pl
from jax.experimental.pallas import tpu as pltpu
from jax.experimental.pallas import tpu_sc as plsc

H = 24
W = 24
NUM_HEADS = 16
NP = H * W
TAB = (2 * H - 1) * (2 * W - 1)
TAB_PAD = 2224
NC, NS, L = 2, 16, 16
ROWS_PER_WORKER = NP // 2
CHUNK = 16
NCHUNK = ROWS_PER_WORKER // CHUNK
JB = NP // L


def _sc_body(table_hbm, idx_hbm, top_hbm, p2c_hbm, out_hbm,
             table_v, col_v, p2c_v, top_v,
             idx_v0, idx_v1, row_v0, row_v1,
             sem_tab, sem_in0, sem_in1, sem_out0, sem_out1):
    cid = lax.axis_index("c")
    sid = lax.axis_index("s")
    wid = sid * NC + cid
    h = wid // 2
    t = wid % 2
    row0 = t * ROWS_PER_WORKER

    idx_bufs = (idx_v0, idx_v1)
    row_bufs = (row_v0, row_v1)
    sem_ins = (sem_in0, sem_in1)
    sem_outs = (sem_out0, sem_out1)

    lanes = lax.iota(jnp.int32, L)
    zeros = lanes * 0
    h_vec = zeros + h

    tab_cp = pltpu.async_copy(table_hbm, table_v, sem_tab)
    in_cp0 = pltpu.async_copy(
        idx_hbm.at[pl.ds(row0, CHUNK), :], idx_bufs[0], sem_ins[0])
    tab_cp.wait()

    def _compact(k, _):
        g = plsc.load_gather(table_v, [k * L + lanes, h_vec])
        col_v[pl.ds(k * L, L)] = g
        return 0

    lax.fori_loop(0, TAB_PAD // L, _compact, 0, unroll=4)

    pltpu.sync_copy(p2c_hbm.at[h, pl.ds(row0, ROWS_PER_WORKER)], p2c_v)

    @pl.when(t == 0)
    def _():
        pltpu.sync_copy(top_hbm.at[h], top_v)
        pltpu.sync_copy(top_v, out_hbm.at[h, 0, :])

    in_cp0.wait()

    for c in range(NCHUNK):
        b = c % 2
        idx_b = idx_bufs[b]
        row_b = row_bufs[b]
        if c + 1 < NCHUNK:
            nb = (c + 1) % 2
            pltpu.async_copy(
                idx_hbm.at[pl.ds(row0 + (c + 1) * CHUNK, CHUNK), :],
                idx_bufs[nb], sem_ins[nb])
        if c >= 2:
            pltpu.make_async_copy(
                row_b,
                out_hbm.at[h, pl.ds(1 + row0 + (c - 2) * CHUNK, CHUNK), :],
                sem_outs[b]).wait()
        if c >= 1:
            pltpu.make_async_copy(
                idx_hbm.at[pl.ds(row0 + c * CHUNK, CHUNK), :],
                idx_b, sem_ins[b]).wait()

        p2c_vals = plsc.load_gather(p2c_v, [c * CHUNK + lanes])
        plsc.store_scatter(row_b, [lanes, zeros], p2c_vals)

        def row_body(r, _):
            for jb in range(JB):
                iv = idx_b[r, pl.ds(jb * L, L)]
                g = plsc.load_gather(col_v, [iv])
                row_b[r, pl.ds(1 + jb * L, L)] = g
            return 0

        lax.fori_loop(0, CHUNK, row_body, 0)
        pltpu.async_copy(
            row_b,
            out_hbm.at[h, pl.ds(1 + row0 + c * CHUNK, CHUNK), :],
            sem_outs[b])

    for c in (NCHUNK - 2, NCHUNK - 1):
        b = c % 2
        pltpu.make_async_copy(
            row_bufs[b],
            out_hbm.at[h, pl.ds(1 + row0 + c * CHUNK, CHUNK), :],
            sem_outs[b]).wait()


@jax.jit
def kernel(relative_position_bias_table, cls_to_patches, patches_to_cls,
           cls_to_cls, relative_position_index):
    table = jnp.pad(relative_position_bias_table,
                    ((0, TAB_PAD - TAB), (0, 0)))
    idx = relative_position_index.astype(jnp.int32)
    top = jnp.concatenate(
        [cls_to_cls.reshape(NUM_HEADS, 1),
         cls_to_patches.reshape(NUM_HEADS, NP)], axis=1)
    p2c = patches_to_cls.reshape(NUM_HEADS, NP)

    mesh = plsc.VectorSubcoreMesh(core_axis_name="c", subcore_axis_name="s")
    run = pl.kernel(
        _sc_body,
        out_type=jax.ShapeDtypeStruct((NUM_HEADS, NP + 1, NP + 1),
                                      jnp.float32),
        mesh=mesh,
        compiler_params=pltpu.CompilerParams(use_tc_tiling_on_sc=False,
                                              needs_layout_passes=False),
        scratch_types=[
            pltpu.VMEM((TAB_PAD, NUM_HEADS), jnp.float32),
            pltpu.VMEM((TAB_PAD,), jnp.float32),
            pltpu.VMEM((ROWS_PER_WORKER,), jnp.float32),
            pltpu.VMEM((NP + 1,), jnp.float32),
            pltpu.VMEM((CHUNK, NP), jnp.int32),
            pltpu.VMEM((CHUNK, NP), jnp.int32),
            pltpu.VMEM((CHUNK, NP + 1), jnp.float32),
            pltpu.VMEM((CHUNK, NP + 1), jnp.float32),
            pltpu.SemaphoreType.DMA,
            pltpu.SemaphoreType.DMA,
            pltpu.SemaphoreType.DMA,
            pltpu.SemaphoreType.DMA,
            pltpu.SemaphoreType.DMA,
        ],
    )
    out = run(table, idx, top, p2c)
    return out[None, ...]


# speedup vs baseline: 5.3581x; 1.0740x over previous
import jax
import jax.numpy as jnp
from jax import lax
from jax.experimental import pallas as pl
from jax.experimental.pallas import tpu as pltpu
from jax.experimental.pallas import tpu_sc as plsc

H = 24
W = 24
NUM_HEADS = 16
NP = H * W
TAB = (2 * H - 1) * (2 * W - 1)
TAB_PAD = 2224
NC, NS, L = 2, 16, 16
ROWS_PER_WORKER = NP // 2
CHUNK = 16
NCHUNK = ROWS_PER_WORKER // CHUNK
JB = NP // L


def _sc_body(table_hbm, idx_hbm, top_hbm, p2c_hbm, out_hbm,
             table_v, col_v, p2c_v, top_v,
             idx_v0, idx_v1, row_v0, row_v1,
             sem_tab, sem_in0, sem_in1, sem_out0, sem_out1):
    cid = lax.axis_index("c")
    sid = lax.axis_index("s")
    wid = sid * NC + cid
    h = wid // 2
    t = wid % 2
    row0 = t * ROWS_PER_WORKER

    idx_bufs = (idx_v0, idx_v1)
    row_bufs = (row_v0, row_v1)
    sem_ins = (sem_in0, sem_in1)
    sem_outs = (sem_out0, sem_out1)

    lanes = lax.iota(jnp.int32, L)
    zeros = lanes * 0
    h_vec = zeros + h

    tab_cp = pltpu.async_copy(table_hbm, table_v, sem_tab)
    in_cp0 = pltpu.async_copy(
        idx_hbm.at[pl.ds(row0, CHUNK), :], idx_bufs[0], sem_ins[0])
    tab_cp.wait()

    def _compact(k, _):
        g = plsc.load_gather(table_v, [k * L + lanes, h_vec])
        col_v[pl.ds(k * L, L)] = g
        return 0

    lax.fori_loop(0, TAB_PAD // L, _compact, 0, unroll=4)

    pltpu.sync_copy(p2c_hbm.at[h, pl.ds(row0, ROWS_PER_WORKER)], p2c_v)

    @pl.when(t == 0)
    def _():
        pltpu.sync_copy(top_hbm.at[h], top_v)
        pltpu.sync_copy(top_v, out_hbm.at[0, h, 0, :])

    in_cp0.wait()

    for c in range(NCHUNK):
        b = c % 2
        idx_b = idx_bufs[b]
        row_b = row_bufs[b]
        if c + 1 < NCHUNK:
            nb = (c + 1) % 2
            pltpu.async_copy(
                idx_hbm.at[pl.ds(row0 + (c + 1) * CHUNK, CHUNK), :],
                idx_bufs[nb], sem_ins[nb])
        if c >= 2:
            pltpu.make_async_copy(
                row_b,
                out_hbm.at[0, h, pl.ds(1 + row0 + (c - 2) * CHUNK, CHUNK), :],
                sem_outs[b]).wait()
        if c >= 1:
            pltpu.make_async_copy(
                idx_hbm.at[pl.ds(row0 + c * CHUNK, CHUNK), :],
                idx_b, sem_ins[b]).wait()

        p2c_vals = plsc.load_gather(p2c_v, [c * CHUNK + lanes])
        plsc.store_scatter(row_b, [lanes, zeros], p2c_vals)

        def row_body(r, _):
            for jb in range(JB):
                iv = idx_b[r, pl.ds(jb * L, L)]
                g = plsc.load_gather(col_v, [iv])
                row_b[r, pl.ds(1 + jb * L, L)] = g
            return 0

        lax.fori_loop(0, CHUNK, row_body, 0)
        pltpu.async_copy(
            row_b,
            out_hbm.at[0, h, pl.ds(1 + row0 + c * CHUNK, CHUNK), :],
            sem_outs[b])

    for c in (NCHUNK - 2, NCHUNK - 1):
        b = c % 2
        pltpu.make_async_copy(
            row_bufs[b],
            out_hbm.at[0, h, pl.ds(1 + row0 + c * CHUNK, CHUNK), :],
            sem_outs[b]).wait()


@jax.jit
def kernel(relative_position_bias_table, cls_to_patches, patches_to_cls,
           cls_to_cls, relative_position_index):
    table = jnp.pad(relative_position_bias_table,
                    ((0, TAB_PAD - TAB), (0, 0)))
    idx = relative_position_index.astype(jnp.int32)
    top = jnp.concatenate(
        [cls_to_cls.reshape(NUM_HEADS, 1),
         cls_to_patches.reshape(NUM_HEADS, NP)], axis=1)
    p2c = patches_to_cls.reshape(NUM_HEADS, NP)

    mesh = plsc.VectorSubcoreMesh(core_axis_name="c", subcore_axis_name="s")
    run = pl.kernel(
        _sc_body,
        out_type=jax.ShapeDtypeStruct((1, NUM_HEADS, NP + 1, NP + 1),
                                      jnp.float32),
        mesh=mesh,
        compiler_params=pltpu.CompilerParams(use_tc_tiling_on_sc=False,
                                              needs_layout_passes=False),
        scratch_types=[
            pltpu.VMEM((TAB_PAD, NUM_HEADS), jnp.float32),
            pltpu.VMEM((TAB_PAD,), jnp.float32),
            pltpu.VMEM((ROWS_PER_WORKER,), jnp.float32),
            pltpu.VMEM((NP + 1,), jnp.float32),
            pltpu.VMEM((CHUNK, NP), jnp.int32),
            pltpu.VMEM((CHUNK, NP), jnp.int32),
            pltpu.VMEM((CHUNK, NP + 1), jnp.float32),
            pltpu.VMEM((CHUNK, NP + 1), jnp.float32),
            pltpu.SemaphoreType.DMA,
            pltpu.SemaphoreType.DMA,
            pltpu.SemaphoreType.DMA,
            pltpu.SemaphoreType.DMA,
            pltpu.SemaphoreType.DMA,
        ],
    )
    return run(table, idx, top, p2c)


# speedup vs baseline: 6.9744x; 1.3016x over previous
import jax
import jax.numpy as jnp
from jax import lax
from jax.experimental import pallas as pl
from jax.experimental.pallas import tpu as pltpu
from jax.experimental.pallas import tpu_sc as plsc

H = 24
W = 24
NUM_HEADS = 16
NP = H * W
TAB = (2 * H - 1) * (2 * W - 1)
TAB_PAD = 2240
NC, NS, L = 2, 16, 16
CHUNK = 24
NCH = 12
JB = NP // L
TR = 73
TCOLS = 5
GSZ = TCOLS * 8 * 128
HEAD_SZ = TR * GSZ
P2C_LEN = 304


def _sc_body(table_hbm, idx_hbm, top_hbm, p2c_hbm, out_hbm,
             table_v, col_v, p2c_v, top_v,
             idx_v0, idx_v1, buf_v0, buf_v1,
             sem_tab, sem_in0, sem_in1, sem_out0, sem_out1):
    cid = lax.axis_index("c")
    sid = lax.axis_index("s")
    wid = sid * NC + cid
    h = wid // 2
    t = wid % 2
    or0 = t * (NCH * CHUNK)

    idx_bufs = (idx_v0, idx_v1)
    bufs = (buf_v0, buf_v1)
    sem_ins = (sem_in0, sem_in1)
    sem_outs = (sem_out0, sem_out1)

    lanes = lax.iota(jnp.int32, L)
    zeros = lanes * 0

    tab_cp = pltpu.async_copy(table_hbm, table_v, sem_tab)
    pltpu.async_copy(idx_hbm.at[pl.ds(or0, CHUNK), :], idx_v0, sem_in0)
    pltpu.async_copy(idx_hbm.at[pl.ds(or0 + CHUNK, CHUNK), :], idx_v1,
                     sem_in1)
    tab_cp.wait()

    def _compact(k, _):
        g = plsc.load_gather(table_v, [lanes * NUM_HEADS + (k * 256 + h)])
        col_v[pl.ds(k * L, L)] = g
        return 0

    lax.fori_loop(0, TAB_PAD // L, _compact, 0, unroll=4)

    pltpu.sync_copy(p2c_hbm.at[h, pl.ds(or0, P2C_LEN)], p2c_v)

    @pl.when(t == 0)
    def _():
        pltpu.sync_copy(top_hbm.at[h], top_v)

    def gather_rows(idx_b, buf_b, nrows):
        def row_body(r, _):
            base = (jnp.right_shift(r, 3) * GSZ
                    + jnp.bitwise_and(r, 7) * 128)
            for jb in range(JB):
                iv = idx_b[r, pl.ds(jb * L, L)]
                g = plsc.load_gather(col_v, [iv])
                cvec = lanes + (1 + jb * L)
                tv = jnp.right_shift(cvec, 7)
                off = (base + cvec) + tv * 896
                plsc.store_scatter(buf_b, [off], g)
            return 0

        lax.fori_loop(0, nrows, row_body, 0)

    col0_16 = jnp.right_shift(lanes, 3) * GSZ + jnp.bitwise_and(lanes, 7) * 128
    col0_8 = 2 * GSZ + lanes * 128
    m8 = lanes < 8

    for cc in range(NCH):
        b = cc % 2
        idx_b = idx_bufs[b]
        buf_b = bufs[b]
        or_s = or0 + cc * CHUNK
        gr = t * (NCH * 3) + 3 * cc

        pltpu.make_async_copy(
            idx_hbm.at[pl.ds(or_s, CHUNK), :], idx_b, sem_ins[b]).wait()
        if cc >= 2:
            pltpu.make_async_copy(
                bufs[b], out_hbm.at[h, pl.ds((gr - 6) * GSZ, 3 * GSZ)],
                sem_outs[b]).wait()

        v16 = plsc.load_gather(p2c_v, [cc * CHUNK + lanes])
        plsc.store_scatter(buf_b, [col0_16], v16)
        v8 = plsc.load_gather(p2c_v, [cc * CHUNK + L + lanes], mask=m8)
        plsc.store_scatter(buf_b, [col0_8], v8, mask=m8)

        gather_rows(idx_b, buf_b, CHUNK)

        if cc == 0:
            @pl.when(t == 0)
            def _():
                for k in range(37):
                    dst = (k >> 3) * 1024 + (k & 7) * L
                    buf_b[pl.ds(dst, L)] = top_v[pl.ds(k * L, L)]

        pltpu.async_copy(
            buf_b, out_hbm.at[h, pl.ds(gr * GSZ, 3 * GSZ)], sem_outs[b])

        if cc + 2 < NCH:
            pltpu.async_copy(
                idx_hbm.at[pl.ds(or_s + 2 * CHUNK, CHUNK), :],
                idx_b, sem_ins[b])

    @pl.when(t == 1)
    def _():
        pltpu.sync_copy(idx_hbm.at[pl.ds(8 * (TR - 1), 8), :],
                        idx_v0.at[pl.ds(0, 8), :])
        pltpu.make_async_copy(
            buf_v0,
            out_hbm.at[h, pl.ds((t * (NCH * 3) + 3 * (NCH - 2)) * GSZ,
                                3 * GSZ)],
            sem_out0).wait()
        v8t = plsc.load_gather(p2c_v, [NCH * CHUNK + lanes], mask=m8)
        plsc.store_scatter(buf_v0, [lanes * 128], v8t, mask=m8)
        gather_rows(idx_v0, buf_v0, 8)
        pltpu.async_copy(buf_v0.at[pl.ds(0, GSZ)],
                         out_hbm.at[h, pl.ds((TR - 1) * GSZ, GSZ)],
                         sem_out0)
        pltpu.make_async_copy(buf_v0.at[pl.ds(0, GSZ)],
                              out_hbm.at[h, pl.ds((TR - 1) * GSZ, GSZ)],
                              sem_out0).wait()

    @pl.when(t == 0)
    def _():
        gr10 = 3 * (NCH - 2)
        pltpu.make_async_copy(
            buf_v0, out_hbm.at[h, pl.ds(gr10 * GSZ, 3 * GSZ)],
            sem_out0).wait()

    gr11 = t * (NCH * 3) + 3 * (NCH - 1)
    pltpu.make_async_copy(
        buf_v1, out_hbm.at[h, pl.ds(gr11 * GSZ, 3 * GSZ)],
        sem_out1).wait()


@jax.jit
def kernel(relative_position_bias_table, cls_to_patches, patches_to_cls,
           cls_to_cls, relative_position_index):
    table_flat = jnp.pad(relative_position_bias_table,
                         ((0, TAB_PAD - TAB), (0, 0))).reshape(-1)
    idx_pad = jnp.pad(relative_position_index.astype(jnp.int32),
                      ((1, 7), (0, 0)))
    top_pad = jnp.concatenate(
        [cls_to_cls.reshape(NUM_HEADS, 1),
         cls_to_patches.reshape(NUM_HEADS, NP),
         jnp.zeros((NUM_HEADS, 15), jnp.float32)], axis=1)
    p2c_pad = jnp.pad(patches_to_cls.reshape(NUM_HEADS, NP), ((0, 0), (1, 15)))

    mesh = plsc.VectorSubcoreMesh(core_axis_name="c", subcore_axis_name="s")
    run = pl.kernel(
        _sc_body,
        out_type=jax.ShapeDtypeStruct((NUM_HEADS, HEAD_SZ), jnp.float32),
        mesh=mesh,
        compiler_params=pltpu.CompilerParams(use_tc_tiling_on_sc=False,
                                             needs_layout_passes=False),
        scratch_types=[
            pltpu.VMEM((TAB_PAD * NUM_HEADS,), jnp.float32),
            pltpu.VMEM((TAB_PAD,), jnp.float32),
            pltpu.VMEM((P2C_LEN,), jnp.float32),
            pltpu.VMEM((592,), jnp.float32),
            pltpu.VMEM((CHUNK, NP), jnp.int32),
            pltpu.VMEM((CHUNK, NP), jnp.int32),
            pltpu.VMEM((3 * GSZ,), jnp.float32),
            pltpu.VMEM((3 * GSZ,), jnp.float32),
            pltpu.SemaphoreType.DMA,
            pltpu.SemaphoreType.DMA,
            pltpu.SemaphoreType.DMA,
            pltpu.SemaphoreType.DMA,
            pltpu.SemaphoreType.DMA,
        ],
    )
    out = run(table_flat, idx_pad, top_pad, p2c_pad)
    out = out.reshape(NUM_HEADS, TR, TCOLS, 8, 128)
    out = out.transpose(0, 1, 3, 2, 4).reshape(NUM_HEADS, TR * 8,
                                               TCOLS * 128)
    return out[None, :, :NP + 1, :NP + 1]


# speedup vs baseline: 7.0254x; 1.0073x over previous
import jax
import jax.numpy as jnp
from jax import lax
from jax.experimental import pallas as pl
from jax.experimental.pallas import tpu as pltpu
from jax.experimental.pallas import tpu_sc as plsc

H = 24
W = 24
NUM_HEADS = 16
NP = H * W
TAB = (2 * H - 1) * (2 * W - 1)
TAB_PAD = 2240
NC, NS, L = 2, 16, 16
CHUNK = 24
NCH = 12
JB = NP // L
TR = 73
TCOLS = 5
GSZ = TCOLS * 8 * 128
HEAD_SZ = TR * GSZ
P2C_LEN = 304


def _sc_body(table_hbm, idx_hbm, top_hbm, p2c_hbm, out_hbm,
             table_v, col_v, p2c_v, top_v,
             idx_v0, idx_v1, buf_v0, buf_v1,
             sem_tab, sem_in0, sem_in1, sem_out0, sem_out1):
    cid = lax.axis_index("c")
    sid = lax.axis_index("s")
    wid = sid * NC + cid
    h = wid // 2
    t = wid % 2
    or0 = t * (NCH * CHUNK)

    idx_bufs = (idx_v0, idx_v1)
    bufs = (buf_v0, buf_v1)
    sem_ins = (sem_in0, sem_in1)
    sem_outs = (sem_out0, sem_out1)

    lanes = lax.iota(jnp.int32, L)
    zeros = lanes * 0

    tab_cp = pltpu.async_copy(table_hbm, table_v, sem_tab)
    pltpu.async_copy(idx_hbm.at[pl.ds(or0, CHUNK), :], idx_v0, sem_in0)
    pltpu.async_copy(idx_hbm.at[pl.ds(or0 + CHUNK, CHUNK), :], idx_v1,
                     sem_in1)
    tab_cp.wait()

    def _compact(k, _):
        g = plsc.load_gather(table_v, [lanes * NUM_HEADS + (k * 256 + h)])
        col_v[pl.ds(k * L, L)] = g
        return 0

    lax.fori_loop(0, TAB_PAD // L, _compact, 0, unroll=4)

    pltpu.sync_copy(p2c_hbm.at[h, pl.ds(or0, P2C_LEN)], p2c_v)

    @pl.when(t == 0)
    def _():
        pltpu.sync_copy(top_hbm.at[h], top_v)

    def gather_rows(idx_b, buf_b, nrows):
        def row_body(r, _):
            base = (jnp.right_shift(r, 3) * GSZ
                    + jnp.bitwise_and(r, 7) * 128)
            for jb in range(JB):
                iv = idx_b[r, pl.ds(jb * L, L)]
                g = plsc.load_gather(col_v, [iv])
                c0 = 1 + jb * L
                if (c0 % 128) + L <= 128:
                    buf_b[pl.ds(base + (c0 + 896 * (c0 // 128)), L)] = g
                else:
                    cvec = lanes + c0
                    tv = jnp.right_shift(cvec, 7)
                    off = (base + cvec) + tv * 896
                    plsc.store_scatter(buf_b, [off], g)
            return 0

        lax.fori_loop(0, nrows, row_body, 0)

    col0_16 = jnp.right_shift(lanes, 3) * GSZ + jnp.bitwise_and(lanes, 7) * 128
    col0_8 = 2 * GSZ + lanes * 128
    m8 = lanes < 8

    for cc in range(NCH):
        b = cc % 2
        idx_b = idx_bufs[b]
        buf_b = bufs[b]
        or_s = or0 + cc * CHUNK
        gr = t * (NCH * 3) + 3 * cc

        pltpu.make_async_copy(
            idx_hbm.at[pl.ds(or_s, CHUNK), :], idx_b, sem_ins[b]).wait()
        if cc >= 2:
            pltpu.make_async_copy(
                bufs[b], out_hbm.at[h, pl.ds((gr - 6) * GSZ, 3 * GSZ)],
                sem_outs[b]).wait()

        v16 = plsc.load_gather(p2c_v, [cc * CHUNK + lanes])
        plsc.store_scatter(buf_b, [col0_16], v16)
        v8 = plsc.load_gather(p2c_v, [cc * CHUNK + L + lanes], mask=m8)
        plsc.store_scatter(buf_b, [col0_8], v8, mask=m8)

        gather_rows(idx_b, buf_b, CHUNK)

        if cc == 0:
            @pl.when(t == 0)
            def _():
                for k in range(37):
                    dst = (k >> 3) * 1024 + (k & 7) * L
                    buf_b[pl.ds(dst, L)] = top_v[pl.ds(k * L, L)]

        pltpu.async_copy(
            buf_b, out_hbm.at[h, pl.ds(gr * GSZ, 3 * GSZ)], sem_outs[b])

        if cc + 2 < NCH:
            pltpu.async_copy(
                idx_hbm.at[pl.ds(or_s + 2 * CHUNK, CHUNK), :],
                idx_b, sem_ins[b])

    @pl.when(t == 1)
    def _():
        pltpu.sync_copy(idx_hbm.at[pl.ds(8 * (TR - 1), 8), :],
                        idx_v0.at[pl.ds(0, 8), :])
        pltpu.make_async_copy(
            buf_v0,
            out_hbm.at[h, pl.ds((t * (NCH * 3) + 3 * (NCH - 2)) * GSZ,
                                3 * GSZ)],
            sem_out0).wait()
        v8t = plsc.load_gather(p2c_v, [NCH * CHUNK + lanes], mask=m8)
        plsc.store_scatter(buf_v0, [lanes * 128], v8t, mask=m8)
        gather_rows(idx_v0, buf_v0, 8)
        pltpu.async_copy(buf_v0.at[pl.ds(0, GSZ)],
                         out_hbm.at[h, pl.ds((TR - 1) * GSZ, GSZ)],
                         sem_out0)
        pltpu.make_async_copy(buf_v0.at[pl.ds(0, GSZ)],
                              out_hbm.at[h, pl.ds((TR - 1) * GSZ, GSZ)],
                              sem_out0).wait()

    @pl.when(t == 0)
    def _():
        gr10 = 3 * (NCH - 2)
        pltpu.make_async_copy(
            buf_v0, out_hbm.at[h, pl.ds(gr10 * GSZ, 3 * GSZ)],
            sem_out0).wait()

    gr11 = t * (NCH * 3) + 3 * (NCH - 1)
    pltpu.make_async_copy(
        buf_v1, out_hbm.at[h, pl.ds(gr11 * GSZ, 3 * GSZ)],
        sem_out1).wait()


@jax.jit
def kernel(relative_position_bias_table, cls_to_patches, patches_to_cls,
           cls_to_cls, relative_position_index):
    table_flat = jnp.pad(relative_position_bias_table,
                         ((0, TAB_PAD - TAB), (0, 0))).reshape(-1)
    idx_pad = jnp.pad(relative_position_index.astype(jnp.int32),
                      ((1, 7), (0, 0)))
    top_pad = jnp.concatenate(
        [cls_to_cls.reshape(NUM_HEADS, 1),
         cls_to_patches.reshape(NUM_HEADS, NP),
         jnp.zeros((NUM_HEADS, 15), jnp.float32)], axis=1)
    p2c_pad = jnp.pad(patches_to_cls.reshape(NUM_HEADS, NP), ((0, 0), (1, 15)))

    mesh = plsc.VectorSubcoreMesh(core_axis_name="c", subcore_axis_name="s")
    run = pl.kernel(
        _sc_body,
        out_type=jax.ShapeDtypeStruct((NUM_HEADS, HEAD_SZ), jnp.float32),
        mesh=mesh,
        compiler_params=pltpu.CompilerParams(use_tc_tiling_on_sc=False,
                                             needs_layout_passes=False),
        scratch_types=[
            pltpu.VMEM((TAB_PAD * NUM_HEADS,), jnp.float32),
            pltpu.VMEM((TAB_PAD,), jnp.float32),
            pltpu.VMEM((P2C_LEN,), jnp.float32),
            pltpu.VMEM((592,), jnp.float32),
            pltpu.VMEM((CHUNK, NP), jnp.int32),
            pltpu.VMEM((CHUNK, NP), jnp.int32),
            pltpu.VMEM((3 * GSZ,), jnp.float32),
            pltpu.VMEM((3 * GSZ,), jnp.float32),
            pltpu.SemaphoreType.DMA,
            pltpu.SemaphoreType.DMA,
            pltpu.SemaphoreType.DMA,
            pltpu.SemaphoreType.DMA,
            pltpu.SemaphoreType.DMA,
        ],
    )
    out = run(table_flat, idx_pad, top_pad, p2c_pad)
    out = out.reshape(NUM_HEADS, TR, TCOLS, 8, 128)
    out = out.transpose(0, 1, 3, 2, 4).reshape(NUM_HEADS, TR * 8,
                                               TCOLS * 128)
    return out[None, :, :NP + 1, :NP + 1]


# speedup vs baseline: 10.7577x; 1.5313x over previous
import jax
import jax.numpy as jnp
from jax import lax
from jax.experimental import pallas as pl
from jax.experimental.pallas import tpu as pltpu
from jax.experimental.pallas import tpu_sc as plsc

H = 24
W = 24
NUM_HEADS = 16
NP = H * W
TAB = (2 * H - 1) * (2 * W - 1)
TAB_PAD = 2240
NC, NS, L = 2, 16, 16
CHUNK = 24
NCH = 12
JB = NP // L
TR = 73
TCOLS = 5
GSZ = TCOLS * 8 * 128
HEAD_SZ = TR * GSZ
P2C_LEN = 304


def _sc_body(table_hbm, idx_hbm, top_hbm, p2c_hbm, out_hbm,
             table_v, col_v, p2c_v, top_v,
             idx_v0, idx_v1, buf_v0, buf_v1,
             sem_tab, sem_in0, sem_in1, sem_out0, sem_out1):
    cid = lax.axis_index("c")
    sid = lax.axis_index("s")
    wid = sid * NC + cid
    h = wid // 2
    t = wid % 2
    or0 = t * (NCH * CHUNK)

    idx_bufs = (idx_v0, idx_v1)
    bufs = (buf_v0, buf_v1)
    sem_ins = (sem_in0, sem_in1)
    sem_outs = (sem_out0, sem_out1)

    lanes = lax.iota(jnp.int32, L)
    zeros = lanes * 0

    tab_cp = pltpu.async_copy(table_hbm, table_v, sem_tab)
    pltpu.async_copy(idx_hbm.at[pl.ds(or0, CHUNK), :], idx_v0, sem_in0)
    pltpu.async_copy(idx_hbm.at[pl.ds(or0 + CHUNK, CHUNK), :], idx_v1,
                     sem_in1)
    tab_cp.wait()

    def _compact(k, _):
        g = plsc.load_gather(table_v, [lanes * NUM_HEADS + (k * 256 + h)])
        col_v[pl.ds(k * L, L)] = g
        return 0

    lax.fori_loop(0, TAB_PAD // L, _compact, 0, unroll=4)

    pltpu.sync_copy(p2c_hbm.at[h, pl.ds(or0, P2C_LEN)], p2c_v)

    @pl.when(t == 0)
    def _():
        pltpu.sync_copy(top_hbm.at[h], top_v)

    def gather_rows(idx_b, buf_b, nrows):
        def row_body(r, _):
            base = (jnp.right_shift(r, 3) * GSZ
                    + jnp.bitwise_and(r, 7) * 128)
            for jb0 in range(0, JB, 6):
                gs = []
                for jb in range(jb0, jb0 + 6):
                    iv = idx_b[r, pl.ds(jb * L, L)]
                    gs.append(plsc.load_gather(col_v, [iv]))
                for i, jb in enumerate(range(jb0, jb0 + 6)):
                    g = gs[i]
                    c0 = 1 + jb * L
                    if (c0 % 128) + L <= 128:
                        buf_b[pl.ds(base + (c0 + 896 * (c0 // 128)), L)] = g
                    else:
                        cvec = lanes + c0
                        tv = jnp.right_shift(cvec, 7)
                        off = (base + cvec) + tv * 896
                        plsc.store_scatter(buf_b, [off], g)
            return 0

        lax.fori_loop(0, nrows, row_body, 0)

    col0_16 = jnp.right_shift(lanes, 3) * GSZ + jnp.bitwise_and(lanes, 7) * 128
    col0_8 = 2 * GSZ + lanes * 128
    m8 = lanes < 8

    for cc in range(NCH):
        b = cc % 2
        idx_b = idx_bufs[b]
        buf_b = bufs[b]
        or_s = or0 + cc * CHUNK
        gr = t * (NCH * 3) + 3 * cc

        pltpu.make_async_copy(
            idx_hbm.at[pl.ds(or_s, CHUNK), :], idx_b, sem_ins[b]).wait()
        if cc >= 2:
            pltpu.make_async_copy(
                bufs[b], out_hbm.at[h, pl.ds((gr - 6) * GSZ, 3 * GSZ)],
                sem_outs[b]).wait()

        v16 = plsc.load_gather(p2c_v, [cc * CHUNK + lanes])
        plsc.store_scatter(buf_b, [col0_16], v16)
        v8 = plsc.load_gather(p2c_v, [cc * CHUNK + L + lanes], mask=m8)
        plsc.store_scatter(buf_b, [col0_8], v8, mask=m8)

        gather_rows(idx_b, buf_b, CHUNK)

        if cc == 0:
            @pl.when(t == 0)
            def _():
                for k in range(37):
                    dst = (k >> 3) * 1024 + (k & 7) * L
                    buf_b[pl.ds(dst, L)] = top_v[pl.ds(k * L, L)]

        pltpu.async_copy(
            buf_b, out_hbm.at[h, pl.ds(gr * GSZ, 3 * GSZ)], sem_outs[b])

        if cc + 2 < NCH:
            pltpu.async_copy(
                idx_hbm.at[pl.ds(or_s + 2 * CHUNK, CHUNK), :],
                idx_b, sem_ins[b])

    @pl.when(t == 1)
    def _():
        pltpu.sync_copy(idx_hbm.at[pl.ds(8 * (TR - 1), 8), :],
                        idx_v0.at[pl.ds(0, 8), :])
        pltpu.make_async_copy(
            buf_v0,
            out_hbm.at[h, pl.ds((t * (NCH * 3) + 3 * (NCH - 2)) * GSZ,
                                3 * GSZ)],
            sem_out0).wait()
        v8t = plsc.load_gather(p2c_v, [NCH * CHUNK + lanes], mask=m8)
        plsc.store_scatter(buf_v0, [lanes * 128], v8t, mask=m8)
        gather_rows(idx_v0, buf_v0, 8)
        pltpu.async_copy(buf_v0.at[pl.ds(0, GSZ)],
                         out_hbm.at[h, pl.ds((TR - 1) * GSZ, GSZ)],
                         sem_out0)
        pltpu.make_async_copy(buf_v0.at[pl.ds(0, GSZ)],
                              out_hbm.at[h, pl.ds((TR - 1) * GSZ, GSZ)],
                              sem_out0).wait()

    @pl.when(t == 0)
    def _():
        gr10 = 3 * (NCH - 2)
        pltpu.make_async_copy(
            buf_v0, out_hbm.at[h, pl.ds(gr10 * GSZ, 3 * GSZ)],
            sem_out0).wait()

    gr11 = t * (NCH * 3) + 3 * (NCH - 1)
    pltpu.make_async_copy(
        buf_v1, out_hbm.at[h, pl.ds(gr11 * GSZ, 3 * GSZ)],
        sem_out1).wait()


@jax.jit
def kernel(relative_position_bias_table, cls_to_patches, patches_to_cls,
           cls_to_cls, relative_position_index):
    table_flat = jnp.pad(relative_position_bias_table,
                         ((0, TAB_PAD - TAB), (0, 0))).reshape(-1)
    idx_pad = jnp.pad(relative_position_index.astype(jnp.int32),
                      ((1, 7), (0, 0)))
    top_pad = jnp.concatenate(
        [cls_to_cls.reshape(NUM_HEADS, 1),
         cls_to_patches.reshape(NUM_HEADS, NP),
         jnp.zeros((NUM_HEADS, 15), jnp.float32)], axis=1)
    p2c_pad = jnp.pad(patches_to_cls.reshape(NUM_HEADS, NP), ((0, 0), (1, 15)))

    mesh = plsc.VectorSubcoreMesh(core_axis_name="c", subcore_axis_name="s")
    run = pl.kernel(
        _sc_body,
        out_type=jax.ShapeDtypeStruct((NUM_HEADS, HEAD_SZ), jnp.float32),
        mesh=mesh,
        compiler_params=pltpu.CompilerParams(use_tc_tiling_on_sc=False,
                                             needs_layout_passes=False),
        scratch_types=[
            pltpu.VMEM((TAB_PAD * NUM_HEADS,), jnp.float32),
            pltpu.VMEM((TAB_PAD,), jnp.float32),
            pltpu.VMEM((P2C_LEN,), jnp.float32),
            pltpu.VMEM((592,), jnp.float32),
            pltpu.VMEM((CHUNK, NP), jnp.int32),
            pltpu.VMEM((CHUNK, NP), jnp.int32),
            pltpu.VMEM((3 * GSZ,), jnp.float32),
            pltpu.VMEM((3 * GSZ,), jnp.float32),
            pltpu.SemaphoreType.DMA,
            pltpu.SemaphoreType.DMA,
            pltpu.SemaphoreType.DMA,
            pltpu.SemaphoreType.DMA,
            pltpu.SemaphoreType.DMA,
        ],
    )
    out = run(table_flat, idx_pad, top_pad, p2c_pad)
    out = out.reshape(NUM_HEADS, TR, TCOLS, 8, 128)
    out = out.transpose(0, 1, 3, 2, 4).reshape(NUM_HEADS, TR * 8,
                                               TCOLS * 128)
    return out[None, :, :NP + 1, :NP + 1]


# speedup vs baseline: 10.9744x; 1.0201x over previous
import jax
import jax.numpy as jnp
from jax import lax
from jax.experimental import pallas as pl
from jax.experimental.pallas import tpu as pltpu
from jax.experimental.pallas import tpu_sc as plsc

H = 24
W = 24
NUM_HEADS = 16
NP = H * W
TAB = (2 * H - 1) * (2 * W - 1)
TAB_PAD = 2240
NC, NS, L = 2, 16, 16
CHUNK = 24
NCH = 12
JB = NP // L
TR = 73
TCOLS = 5
GSZ = TCOLS * 8 * 128
HEAD_SZ = TR * GSZ
P2C_LEN = 304


def _sc_body(table_hbm, idx_hbm, top_hbm, p2c_hbm, out_hbm,
             table_v, col_v, p2c_v, top_v,
             idx_v0, idx_v1, buf_v0, buf_v1,
             sem_tab, sem_in0, sem_in1, sem_out0, sem_out1):
    cid = lax.axis_index("c")
    sid = lax.axis_index("s")
    wid = sid * NC + cid
    h = wid // 2
    t = wid % 2
    or0 = t * (NCH * CHUNK)

    idx_bufs = (idx_v0, idx_v1)
    bufs = (buf_v0, buf_v1)
    sem_ins = (sem_in0, sem_in1)
    sem_outs = (sem_out0, sem_out1)

    lanes = lax.iota(jnp.int32, L)
    zeros = lanes * 0

    tab_cp = pltpu.async_copy(table_hbm, table_v, sem_tab)
    pltpu.async_copy(idx_hbm.at[pl.ds(or0, CHUNK), :], idx_v0, sem_in0)
    pltpu.async_copy(idx_hbm.at[pl.ds(or0 + CHUNK, CHUNK), :], idx_v1,
                     sem_in1)
    tab_cp.wait()

    def _compact(k, _):
        g = plsc.load_gather(table_v, [lanes * NUM_HEADS + (k * 256 + h)])
        col_v[pl.ds(k * L, L)] = g
        return 0

    lax.fori_loop(0, TAB_PAD // L, _compact, 0, unroll=4)

    pltpu.sync_copy(p2c_hbm.at[h, pl.ds(or0, P2C_LEN)], p2c_v)

    @pl.when(t == 0)
    def _():
        pltpu.sync_copy(top_hbm.at[h], top_v)

    def gather_rows(idx_b, buf_b, nrows):
        def row_body(r, _):
            base = (jnp.right_shift(r, 3) * GSZ
                    + jnp.bitwise_and(r, 7) * 128)
            for jb0 in range(0, JB, 12):
                gs = []
                for jb in range(jb0, jb0 + 12):
                    iv = idx_b[r, pl.ds(jb * L, L)]
                    gs.append(plsc.load_gather(col_v, [iv]))
                for i, jb in enumerate(range(jb0, jb0 + 12)):
                    g = gs[i]
                    c0 = 1 + jb * L
                    if (c0 % 128) + L <= 128:
                        buf_b[pl.ds(base + (c0 + 896 * (c0 // 128)), L)] = g
                    else:
                        cvec = lanes + c0
                        tv = jnp.right_shift(cvec, 7)
                        off = (base + cvec) + tv * 896
                        plsc.store_scatter(buf_b, [off], g)
            return 0

        lax.fori_loop(0, nrows, row_body, 0)

    col0_16 = jnp.right_shift(lanes, 3) * GSZ + jnp.bitwise_and(lanes, 7) * 128
    col0_8 = 2 * GSZ + lanes * 128
    m8 = lanes < 8

    for cc in range(NCH):
        b = cc % 2
        idx_b = idx_bufs[b]
        buf_b = bufs[b]
        or_s = or0 + cc * CHUNK
        gr = t * (NCH * 3) + 3 * cc

        pltpu.make_async_copy(
            idx_hbm.at[pl.ds(or_s, CHUNK), :], idx_b, sem_ins[b]).wait()
        if cc >= 2:
            pltpu.make_async_copy(
                bufs[b], out_hbm.at[h, pl.ds((gr - 6) * GSZ, 3 * GSZ)],
                sem_outs[b]).wait()

        v16 = plsc.load_gather(p2c_v, [cc * CHUNK + lanes])
        plsc.store_scatter(buf_b, [col0_16], v16)
        v8 = plsc.load_gather(p2c_v, [cc * CHUNK + L + lanes], mask=m8)
        plsc.store_scatter(buf_b, [col0_8], v8, mask=m8)

        gather_rows(idx_b, buf_b, CHUNK)

        if cc == 0:
            @pl.when(t == 0)
            def _():
                for k in range(37):
                    dst = (k >> 3) * 1024 + (k & 7) * L
                    buf_b[pl.ds(dst, L)] = top_v[pl.ds(k * L, L)]

        pltpu.async_copy(
            buf_b, out_hbm.at[h, pl.ds(gr * GSZ, 3 * GSZ)], sem_outs[b])

        if cc + 2 < NCH:
            pltpu.async_copy(
                idx_hbm.at[pl.ds(or_s + 2 * CHUNK, CHUNK), :],
                idx_b, sem_ins[b])

    @pl.when(t == 1)
    def _():
        pltpu.sync_copy(idx_hbm.at[pl.ds(8 * (TR - 1), 8), :],
                        idx_v0.at[pl.ds(0, 8), :])
        pltpu.make_async_copy(
            buf_v0,
            out_hbm.at[h, pl.ds((t * (NCH * 3) + 3 * (NCH - 2)) * GSZ,
                                3 * GSZ)],
            sem_out0).wait()
        v8t = plsc.load_gather(p2c_v, [NCH * CHUNK + lanes], mask=m8)
        plsc.store_scatter(buf_v0, [lanes * 128], v8t, mask=m8)
        gather_rows(idx_v0, buf_v0, 8)
        pltpu.async_copy(buf_v0.at[pl.ds(0, GSZ)],
                         out_hbm.at[h, pl.ds((TR - 1) * GSZ, GSZ)],
                         sem_out0)
        pltpu.make_async_copy(buf_v0.at[pl.ds(0, GSZ)],
                              out_hbm.at[h, pl.ds((TR - 1) * GSZ, GSZ)],
                              sem_out0).wait()

    @pl.when(t == 0)
    def _():
        gr10 = 3 * (NCH - 2)
        pltpu.make_async_copy(
            buf_v0, out_hbm.at[h, pl.ds(gr10 * GSZ, 3 * GSZ)],
            sem_out0).wait()

    gr11 = t * (NCH * 3) + 3 * (NCH - 1)
    pltpu.make_async_copy(
        buf_v1, out_hbm.at[h, pl.ds(gr11 * GSZ, 3 * GSZ)],
        sem_out1).wait()


@jax.jit
def kernel(relative_position_bias_table, cls_to_patches, patches_to_cls,
           cls_to_cls, relative_position_index):
    table_flat = jnp.pad(relative_position_bias_table,
                         ((0, TAB_PAD - TAB), (0, 0))).reshape(-1)
    idx_pad = jnp.pad(relative_position_index.astype(jnp.int32),
                      ((1, 7), (0, 0)))
    top_pad = jnp.concatenate(
        [cls_to_cls.reshape(NUM_HEADS, 1),
         cls_to_patches.reshape(NUM_HEADS, NP),
         jnp.zeros((NUM_HEADS, 15), jnp.float32)], axis=1)
    p2c_pad = jnp.pad(patches_to_cls.reshape(NUM_HEADS, NP), ((0, 0), (1, 15)))

    mesh = plsc.VectorSubcoreMesh(core_axis_name="c", subcore_axis_name="s")
    run = pl.kernel(
        _sc_body,
        out_type=jax.ShapeDtypeStruct((NUM_HEADS, HEAD_SZ), jnp.float32),
        mesh=mesh,
        compiler_params=pltpu.CompilerParams(use_tc_tiling_on_sc=False,
                                             needs_layout_passes=False),
        scratch_types=[
            pltpu.VMEM((TAB_PAD * NUM_HEADS,), jnp.float32),
            pltpu.VMEM((TAB_PAD,), jnp.float32),
            pltpu.VMEM((P2C_LEN,), jnp.float32),
            pltpu.VMEM((592,), jnp.float32),
            pltpu.VMEM((CHUNK, NP), jnp.int32),
            pltpu.VMEM((CHUNK, NP), jnp.int32),
            pltpu.VMEM((3 * GSZ,), jnp.float32),
            pltpu.VMEM((3 * GSZ,), jnp.float32),
            pltpu.SemaphoreType.DMA,
            pltpu.SemaphoreType.DMA,
            pltpu.SemaphoreType.DMA,
            pltpu.SemaphoreType.DMA,
            pltpu.SemaphoreType.DMA,
        ],
    )
    out = run(table_flat, idx_pad, top_pad, p2c_pad)
    out = out.reshape(NUM_HEADS, TR, TCOLS, 8, 128)
    out = out.transpose(0, 1, 3, 2, 4).reshape(NUM_HEADS, TR * 8,
                                               TCOLS * 128)
    return out[None, :, :NP + 1, :NP + 1]
